# baseline (speedup 1.0000x reference)
import functools

import jax
import jax.numpy as jnp
from jax import lax
from jax.experimental import pallas as pl
from jax.experimental.pallas import tpu as pltpu
from jax.experimental.pallas import tpu_sc as plsc

_NUM_CORES = 2
_NUM_SUBCORES = 16
_CHUNK = 128


@jax.jit
def _gather(memory, keys):
    B, = keys.shape
    V, D = memory.shape
    nw = _NUM_CORES * _NUM_SUBCORES
    b_per_w = B // nw
    n_chunks = b_per_w // _CHUNK
    out_rows = B * D // 128
    rows_per_w = out_rows // nw
    mesh = plsc.VectorSubcoreMesh(core_axis_name="c", subcore_axis_name="s")

    @functools.partial(
        pl.kernel,
        mesh=mesh,
        compiler_params=pltpu.CompilerParams(use_tc_tiling_on_sc=False),
        out_type=jax.ShapeDtypeStruct((out_rows, 128), jnp.float32),
        scratch_types=[
            pltpu.VMEM((b_per_w,), jnp.int32),
            pltpu.VMEM((b_per_w, D), jnp.float32),
            pltpu.VMEM((rows_per_w, 128), jnp.float32),
            pltpu.SemaphoreType.DMA,
        ],
    )
    def gather_kernel(table_hbm, idx_hbm, out_hbm, idx_v, rows_v, rows2, sem):
        wid = lax.axis_index("s") * _NUM_CORES + lax.axis_index("c")
        base = wid * b_per_w
        pltpu.sync_copy(idx_hbm.at[pl.ds(base, b_per_w)], idx_v)
        copies = []
        for j in range(n_chunks):
            copies.append(
                pltpu.async_copy(
                    table_hbm.at[idx_v.at[pl.ds(j * _CHUNK, _CHUNK)]],
                    rows_v.at[pl.ds(j * _CHUNK, _CHUNK)],
                    sem,
                )
            )
        for c in copies:
            c.wait()

        def repack(q, carry):
            for h in range(D // 16):
                rows2[q, pl.ds(h * 16, 16)] = rows_v[2 * q, pl.ds(h * 16, 16)]
                rows2[q, pl.ds(D + h * 16, 16)] = rows_v[
                    2 * q + 1, pl.ds(h * 16, 16)
                ]
            return carry

        lax.fori_loop(0, rows_per_w, repack, 0)
        pltpu.sync_copy(rows2, out_hbm.at[pl.ds(wid * rows_per_w, rows_per_w)])

    out_packed = gather_kernel(memory, keys)
    return out_packed.reshape(B, D)


def kernel(memory, keys):
    return _gather(memory, keys)


# speedup vs baseline: 1.6725x; 1.6725x over previous
import functools

import jax
import jax.numpy as jnp
from jax import lax
from jax.experimental import pallas as pl
from jax.experimental.pallas import tpu as pltpu
from jax.experimental.pallas import tpu_sc as plsc

_NUM_CORES = 2
_NUM_SUBCORES = 16


@jax.jit
def _gather(memory, keys):
    B, = keys.shape
    V, D = memory.shape
    nw = _NUM_CORES * _NUM_SUBCORES
    b_per_w = B // nw
    n_blocks = b_per_w // 16
    out_rows = B * D // 128
    rows_per_w = out_rows // nw
    mesh = plsc.VectorSubcoreMesh(core_axis_name="c", subcore_axis_name="s")

    @functools.partial(
        pl.kernel,
        mesh=mesh,
        out_type=jax.ShapeDtypeStruct((out_rows, 128), jnp.float32),
        scratch_types=[
            pltpu.VMEM((b_per_w,), jnp.int32),
            pltpu.VMEM((b_per_w, D), jnp.float32),
            pltpu.VMEM((rows_per_w, 128), jnp.float32),
            pltpu.SemaphoreType.DMA,
        ],
    )
    def gather_kernel(table_hbm, idx_hbm, out_hbm, idx_v, rows_v, rows2, sem):
        wid = lax.axis_index("s") * _NUM_CORES + lax.axis_index("c")
        base = wid * b_per_w
        pltpu.sync_copy(idx_hbm.at[pl.ds(base, b_per_w)], idx_v)

        def fire_block(b, carry):
            kvec = idx_v[pl.ds(b * 16, 16)]
            for u in range(16):
                pltpu.async_copy(
                    table_hbm.at[pl.ds(kvec[u], 1), :],
                    rows_v.at[pl.ds(b * 16 + u, 1), :],
                    sem,
                )
            return carry

        lax.fori_loop(0, n_blocks, fire_block, 0)

        def drain_block(b, carry):
            for u in range(16):
                pltpu.make_async_copy(
                    table_hbm.at[pl.ds(0, 1), :],
                    rows_v.at[pl.ds(b * 16 + u, 1), :],
                    sem,
                ).wait()
            return carry

        lax.fori_loop(0, n_blocks, drain_block, 0)

        def repack(q, carry):
            for h in range(D // 16):
                rows2[q, pl.ds(h * 16, 16)] = rows_v[2 * q, pl.ds(h * 16, 16)]
                rows2[q, pl.ds(D + h * 16, 16)] = rows_v[
                    2 * q + 1, pl.ds(h * 16, 16)
                ]
            return carry

        lax.fori_loop(0, rows_per_w, repack, 0)
        pltpu.sync_copy(rows2, out_hbm.at[pl.ds(wid * rows_per_w, rows_per_w)])

    out_packed = gather_kernel(memory, keys)
    return out_packed.reshape(B, D)


def kernel(memory, keys):
    return _gather(memory, keys)
